# Initial kernel scaffold; baseline (speedup 1.0000x reference)
import jax
import jax.numpy as jnp
from jax.experimental import pallas as pl


def kernel(x, edge_index, W, bias, W_gcn, lin_W, lin_b):
    raise NotImplementedError("write your pallas kernel here")


# speedup vs baseline: 16.6363x; 16.6363x over previous
import functools

import jax
import jax.numpy as jnp
from jax import lax
from jax.experimental import pallas as pl
from jax.experimental.pallas import tpu as pltpu
from jax.experimental.pallas import tpu_sc as plsc

N = 10000
E = 160000
D = 256
DH = 128
EPSILON = 0.1
GAMMA = 0.1

NC = 2
NS = 16
NW = NC * NS
EW = E // NW
CH = 125
NCHUNK = EW // CH
NP = 10240
RPT = NP // NS
ZCH = 128
RB = 1000
NBLK = N // RB


def _fill_rows(ref, val):
    r, c = ref.shape
    assert c % 16 == 0

    def body(i, _):
        for j in range(c // 16):
            ref[i, pl.ds(j * 16, 16)] = jnp.full((16,), val, ref.dtype)
        return 0

    lax.fori_loop(0, r, body, 0)


def _sc_deg(eidx, deg_out, ones_v, zero_v, idx_v, deg_sh):
    cid = lax.axis_index("c")
    sid = lax.axis_index("s")
    wid = sid * NC + cid

    _fill_rows(ones_v, 1.0)
    _fill_rows(zero_v, 0.0)
    pltpu.sync_copy(eidx.at[1, wid], idx_v)

    zrow = sid * RPT
    for c in range(RPT // ZCH):
        pltpu.sync_copy(zero_v, deg_sh.at[pl.ds(zrow + c * ZCH, ZCH)])
    plsc.subcore_barrier()

    def add_chunk(j, _):
        pltpu.sync_copy(ones_v, deg_sh.at[idx_v.at[j]], add=True)
        return 0

    lax.fori_loop(0, NCHUNK, add_chunk, 0)
    plsc.subcore_barrier()

    pltpu.sync_copy(deg_sh.at[pl.ds(zrow, RPT)],
                    deg_out.at[cid, pl.ds(zrow, RPT)])


def _sc_edge_acc(h0, h1, eidx, acc_out, isrc_v, idst_v, rows_v, zero_v, acc_sh):
    cid = lax.axis_index("c")
    sid = lax.axis_index("s")
    wid = sid * NC + cid

    pltpu.sync_copy(eidx.at[0, wid], isrc_v)
    pltpu.sync_copy(eidx.at[1, wid], idst_v)
    _fill_rows(zero_v, 0.0)

    zrow = sid * RPT
    for half in range(2):
        table = h0 if half == 0 else h1
        for c in range(RPT // ZCH):
            pltpu.sync_copy(zero_v, acc_sh.at[pl.ds(zrow + c * ZCH, ZCH)])
        plsc.subcore_barrier()

        def move_chunk(j, _):
            pltpu.sync_copy(table.at[isrc_v.at[j]], rows_v)
            pltpu.sync_copy(rows_v, acc_sh.at[idst_v.at[j]], add=True)
            return 0

        lax.fori_loop(0, NCHUNK, move_chunk, 0)
        plsc.subcore_barrier()

        pltpu.sync_copy(acc_sh.at[pl.ds(zrow, RPT)],
                        acc_out.at[cid, half, pl.ds(zrow, RPT)])
        plsc.subcore_barrier()


def _tc_gcn_mm(x_ref, w_ref, o_ref):
    o_ref[...] = lax.dot_general(
        x_ref[...], w_ref[...], (((1,), (1,)), ((), ())),
        preferred_element_type=jnp.float32)


def _tc_scale(h_ref, degp_ref, h0_ref, h1_ref):
    deg = 1.0 + degp_ref[0, :, 0] + degp_ref[1, :, 0]
    dinv = lax.rsqrt(deg)
    hs = h_ref[...] * dinv[:, None]
    h0_ref[...] = hs[:, :DH]
    h1_ref[...] = hs[:, DH:]


def _tc_tail(x_ref, h0_ref, h1_ref, acc_ref, degp_ref, a_ref, bias_ref,
             linw_ref, linb_ref, o_ref):
    deg = 1.0 + degp_ref[0, :, 0] + degp_ref[1, :, 0]
    dinv = lax.rsqrt(deg)
    acc0 = acc_ref[0, 0] + acc_ref[1, 0]
    acc1 = acc_ref[0, 1] + acc_ref[1, 1]
    acc = jnp.concatenate([acc0, acc1], axis=-1)
    hs = jnp.concatenate([h0_ref[...], h1_ref[...]], axis=-1)
    agg = dinv[:, None] * (acc + hs)
    x = x_ref[...]
    pre = lax.dot_general(x, a_ref[...], (((1,), (1,)), ((), ())),
                          preferred_element_type=jnp.float32)
    pre = pre + agg + bias_ref[...]
    x2 = x + EPSILON * jnp.tanh(pre)
    x3 = jnp.where(x2 > 0, x2, jnp.exp(jnp.minimum(x2, 0.0)) - 1.0)
    o_ref[...] = lax.dot_general(x3, linw_ref[...], (((1,), (1,)), ((), ())),
                                 preferred_element_type=jnp.float32) + linb_ref[...]


def kernel(x, edge_index, W, bias, W_gcn, lin_W, lin_b):
    eidx = edge_index.astype(jnp.int32).reshape(2, NW, NCHUNK, CH)
    A = W - W.T - GAMMA * jnp.eye(D, dtype=W.dtype)
    bias2 = bias.reshape(1, D)
    linb2 = lin_b.reshape(1, D)

    mesh = plsc.VectorSubcoreMesh(core_axis_name="c", subcore_axis_name="s",
                                  num_cores=NC, num_subcores=NS)

    deg_kernel = pl.kernel(
        _sc_deg,
        out_type=jax.ShapeDtypeStruct((NC, NP, 16), jnp.float32),
        mesh=mesh,
        scratch_types=[
            pltpu.VMEM((CH, 16), jnp.float32),
            pltpu.VMEM((ZCH, 16), jnp.float32),
            pltpu.VMEM((NCHUNK, CH), jnp.int32),
            pltpu.VMEM_SHARED((NP, 16), jnp.float32),
        ],
    )
    degp = deg_kernel(eidx)

    h = pl.pallas_call(
        _tc_gcn_mm,
        grid=(NBLK,),
        in_specs=[
            pl.BlockSpec((RB, D), lambda i: (i, 0)),
            pl.BlockSpec((D, D), lambda i: (0, 0)),
        ],
        out_specs=pl.BlockSpec((RB, D), lambda i: (i, 0)),
        out_shape=jax.ShapeDtypeStruct((N, D), jnp.float32),
    )(x, W_gcn)

    h0, h1 = pl.pallas_call(
        _tc_scale,
        grid=(NBLK,),
        in_specs=[
            pl.BlockSpec((RB, D), lambda i: (i, 0)),
            pl.BlockSpec((NC, RB, 16), lambda i: (0, i, 0)),
        ],
        out_specs=[
            pl.BlockSpec((RB, DH), lambda i: (i, 0)),
            pl.BlockSpec((RB, DH), lambda i: (i, 0)),
        ],
        out_shape=[
            jax.ShapeDtypeStruct((N, DH), jnp.float32),
            jax.ShapeDtypeStruct((N, DH), jnp.float32),
        ],
    )(h, degp)

    acc_kernel = pl.kernel(
        _sc_edge_acc,
        out_type=jax.ShapeDtypeStruct((NC, 2, NP, DH), jnp.float32),
        mesh=mesh,
        scratch_types=[
            pltpu.VMEM((NCHUNK, CH), jnp.int32),
            pltpu.VMEM((NCHUNK, CH), jnp.int32),
            pltpu.VMEM((CH, DH), jnp.float32),
            pltpu.VMEM((ZCH, DH), jnp.float32),
            pltpu.VMEM_SHARED((NP, DH), jnp.float32),
        ],
    )
    accp = acc_kernel(h0, h1, eidx)

    out = pl.pallas_call(
        _tc_tail,
        grid=(NBLK,),
        in_specs=[
            pl.BlockSpec((RB, D), lambda i: (i, 0)),
            pl.BlockSpec((RB, DH), lambda i: (i, 0)),
            pl.BlockSpec((RB, DH), lambda i: (i, 0)),
            pl.BlockSpec((NC, 2, RB, DH), lambda i: (0, 0, i, 0)),
            pl.BlockSpec((NC, RB, 16), lambda i: (0, i, 0)),
            pl.BlockSpec((D, D), lambda i: (0, 0)),
            pl.BlockSpec((1, D), lambda i: (0, 0)),
            pl.BlockSpec((D, D), lambda i: (0, 0)),
            pl.BlockSpec((1, D), lambda i: (0, 0)),
        ],
        out_specs=pl.BlockSpec((RB, D), lambda i: (i, 0)),
        out_shape=jax.ShapeDtypeStruct((N, D), jnp.float32),
    )(x, h0, h1, accp, degp, A, bias2, lin_W, linb2)

    return out


# speedup vs baseline: 19.0920x; 1.1476x over previous
import functools

import jax
import jax.numpy as jnp
from jax import lax
from jax.experimental import pallas as pl
from jax.experimental.pallas import tpu as pltpu
from jax.experimental.pallas import tpu_sc as plsc

N = 10000
E = 160000
D = 256
DH = 128
EPSILON = 0.1
GAMMA = 0.1

NC = 2
NS = 16
NW = NC * NS
EW = E // NW
CH = 100
NCHUNK = EW // CH
NP = 10240
RPT = NP // NS
ZCH = 64
RB = 1000
NBLK = N // RB


def _fill_rows(ref, val):
    r, c = ref.shape
    assert c % 16 == 0

    def body(i, _):
        for j in range(c // 16):
            ref[i, pl.ds(j * 16, 16)] = jnp.full((16,), val, ref.dtype)
        return 0

    lax.fori_loop(0, r, body, 0)


def _sc_deg(eidx, deg_out, ones_v, idx_v, deg_sh):
    cid = lax.axis_index("c")
    sid = lax.axis_index("s")
    wid = sid * NC + cid

    _fill_rows(ones_v, 0.0)
    pltpu.sync_copy(eidx.at[1, wid], idx_v)

    zrow = sid * RPT
    for c in range(RPT // ZCH):
        pltpu.sync_copy(ones_v.at[pl.ds(0, ZCH)],
                        deg_sh.at[pl.ds(zrow + c * ZCH, ZCH)])
    _fill_rows(ones_v, 1.0)
    plsc.subcore_barrier()

    def add_chunk(j, _):
        pltpu.sync_copy(ones_v, deg_sh.at[idx_v.at[j]], add=True)
        return 0

    lax.fori_loop(0, NCHUNK, add_chunk, 0)
    plsc.subcore_barrier()

    pltpu.sync_copy(deg_sh.at[pl.ds(zrow, RPT)],
                    deg_out.at[cid, pl.ds(zrow, RPT)])


def _sc_edge_acc(h0, h1, eidx, acc_out, isrc_v, idst_v, rows0_v, rows1_v,
                 acc_sh, sem0, sem1):
    cid = lax.axis_index("c")
    sid = lax.axis_index("s")
    wid = sid * NC + cid

    pltpu.sync_copy(eidx.at[0, wid], isrc_v)
    pltpu.sync_copy(eidx.at[1, wid], idst_v)

    rows = (rows0_v, rows1_v)
    sems = (sem0, sem1)
    zrow = sid * RPT
    for half in range(2):
        table = h0 if half == 0 else h1
        _fill_rows(rows0_v, 0.0)
        for c in range(RPT // ZCH):
            pltpu.sync_copy(rows0_v.at[pl.ds(0, ZCH)],
                            acc_sh.at[pl.ds(zrow + c * ZCH, ZCH)])
        plsc.subcore_barrier()

        pltpu.async_copy(table.at[isrc_v.at[0]], rows[0], sems[0])

        def pair(g, _):
            for b in range(2):
                j = 2 * g + b
                nxt = j + 1
                pltpu.make_async_copy(table.at[isrc_v.at[j]],
                                      rows[b], sems[b]).wait()

                @pl.when(nxt < NCHUNK)
                def _():
                    pltpu.async_copy(table.at[isrc_v.at[nxt]],
                                     rows[1 - b], sems[1 - b])

                pltpu.sync_copy(rows[b], acc_sh.at[idst_v.at[j]], add=True)
            return 0

        lax.fori_loop(0, NCHUNK // 2, pair, 0)
        plsc.subcore_barrier()

        pltpu.sync_copy(acc_sh.at[pl.ds(zrow, RPT)],
                        acc_out.at[cid, half, pl.ds(zrow, RPT)])
        plsc.subcore_barrier()


def _tc_gcn_mm(x_ref, w_ref, o_ref):
    o_ref[...] = lax.dot_general(
        x_ref[...], w_ref[...], (((1,), (1,)), ((), ())),
        preferred_element_type=jnp.float32)


def _tc_scale(h_ref, degp_ref, h0_ref, h1_ref):
    deg = 1.0 + degp_ref[0, :, 0] + degp_ref[1, :, 0]
    dinv = lax.rsqrt(deg)
    hs = h_ref[...] * dinv[:, None]
    h0_ref[...] = hs[:, :DH]
    h1_ref[...] = hs[:, DH:]


def _tc_tail(x_ref, h0_ref, h1_ref, acc_ref, degp_ref, a_ref, bias_ref,
             linw_ref, linb_ref, o_ref):
    deg = 1.0 + degp_ref[0, :, 0] + degp_ref[1, :, 0]
    dinv = lax.rsqrt(deg)
    acc0 = acc_ref[0, 0] + acc_ref[1, 0]
    acc1 = acc_ref[0, 1] + acc_ref[1, 1]
    acc = jnp.concatenate([acc0, acc1], axis=-1)
    hs = jnp.concatenate([h0_ref[...], h1_ref[...]], axis=-1)
    agg = dinv[:, None] * (acc + hs)
    x = x_ref[...]
    pre = lax.dot_general(x, a_ref[...], (((1,), (1,)), ((), ())),
                          preferred_element_type=jnp.float32)
    pre = pre + agg + bias_ref[...]
    x2 = x + EPSILON * jnp.tanh(pre)
    x3 = jnp.where(x2 > 0, x2, jnp.exp(jnp.minimum(x2, 0.0)) - 1.0)
    o_ref[...] = lax.dot_general(x3, linw_ref[...], (((1,), (1,)), ((), ())),
                                 preferred_element_type=jnp.float32) + linb_ref[...]


def kernel(x, edge_index, W, bias, W_gcn, lin_W, lin_b):
    eidx = edge_index.astype(jnp.int32).reshape(2, NW, NCHUNK, CH)
    A = W - W.T - GAMMA * jnp.eye(D, dtype=W.dtype)
    bias2 = bias.reshape(1, D)
    linb2 = lin_b.reshape(1, D)

    mesh = plsc.VectorSubcoreMesh(core_axis_name="c", subcore_axis_name="s",
                                  num_cores=NC, num_subcores=NS)

    deg_kernel = pl.kernel(
        _sc_deg,
        out_type=jax.ShapeDtypeStruct((NC, NP, 16), jnp.float32),
        mesh=mesh,
        scratch_types=[
            pltpu.VMEM((CH, 16), jnp.float32),
            pltpu.VMEM((NCHUNK, CH), jnp.int32),
            pltpu.VMEM_SHARED((NP, 16), jnp.float32),
        ],
    )
    degp = deg_kernel(eidx)

    h = pl.pallas_call(
        _tc_gcn_mm,
        grid=(NBLK,),
        in_specs=[
            pl.BlockSpec((RB, D), lambda i: (i, 0)),
            pl.BlockSpec((D, D), lambda i: (0, 0)),
        ],
        out_specs=pl.BlockSpec((RB, D), lambda i: (i, 0)),
        out_shape=jax.ShapeDtypeStruct((N, D), jnp.float32),
    )(x, W_gcn)

    h0, h1 = pl.pallas_call(
        _tc_scale,
        grid=(NBLK,),
        in_specs=[
            pl.BlockSpec((RB, D), lambda i: (i, 0)),
            pl.BlockSpec((NC, RB, 16), lambda i: (0, i, 0)),
        ],
        out_specs=[
            pl.BlockSpec((RB, DH), lambda i: (i, 0)),
            pl.BlockSpec((RB, DH), lambda i: (i, 0)),
        ],
        out_shape=[
            jax.ShapeDtypeStruct((N, DH), jnp.float32),
            jax.ShapeDtypeStruct((N, DH), jnp.float32),
        ],
    )(h, degp)

    acc_kernel = pl.kernel(
        _sc_edge_acc,
        out_type=jax.ShapeDtypeStruct((NC, 2, NP, DH), jnp.float32),
        mesh=mesh,
        scratch_types=[
            pltpu.VMEM((NCHUNK, CH), jnp.int32),
            pltpu.VMEM((NCHUNK, CH), jnp.int32),
            pltpu.VMEM((CH, DH), jnp.float32),
            pltpu.VMEM((CH, DH), jnp.float32),
            pltpu.VMEM_SHARED((NP, DH), jnp.float32),
            pltpu.SemaphoreType.DMA,
            pltpu.SemaphoreType.DMA,
        ],
    )
    accp = acc_kernel(h0, h1, eidx)

    out = pl.pallas_call(
        _tc_tail,
        grid=(NBLK,),
        in_specs=[
            pl.BlockSpec((RB, D), lambda i: (i, 0)),
            pl.BlockSpec((RB, DH), lambda i: (i, 0)),
            pl.BlockSpec((RB, DH), lambda i: (i, 0)),
            pl.BlockSpec((NC, 2, RB, DH), lambda i: (0, 0, i, 0)),
            pl.BlockSpec((NC, RB, 16), lambda i: (0, i, 0)),
            pl.BlockSpec((D, D), lambda i: (0, 0)),
            pl.BlockSpec((1, D), lambda i: (0, 0)),
            pl.BlockSpec((D, D), lambda i: (0, 0)),
            pl.BlockSpec((1, D), lambda i: (0, 0)),
        ],
        out_specs=pl.BlockSpec((RB, D), lambda i: (i, 0)),
        out_shape=jax.ShapeDtypeStruct((N, D), jnp.float32),
    )(x, h0, h1, accp, degp, A, bias2, lin_W, linb2)

    return out


# speedup vs baseline: 20.0703x; 1.0512x over previous
import functools

import jax
import jax.numpy as jnp
from jax import lax
from jax.experimental import pallas as pl
from jax.experimental.pallas import tpu as pltpu
from jax.experimental.pallas import tpu_sc as plsc

N = 10000
E = 160000
D = 256
DH = 128
EPSILON = 0.1
GAMMA = 0.1

NC = 2
NS = 16
NW = NC * NS
EW = E // NW
CH = 100
NCHUNK = EW // CH
NP = 10240
RPT = NP // NS
ZCH = 32
RB = 1000
NBLK = N // RB


def _fill_rows(ref, val):
    r, c = ref.shape
    assert c % 16 == 0

    def body(i, _):
        for j in range(c // 16):
            ref[i, pl.ds(j * 16, 16)] = jnp.full((16,), val, ref.dtype)
        return 0

    lax.fori_loop(0, r, body, 0)


def _sc_deg(eidx, deg_out, ones_v, idx_v, deg_sh):
    cid = lax.axis_index("c")
    sid = lax.axis_index("s")
    wid = sid * NC + cid

    _fill_rows(ones_v, 0.0)
    pltpu.sync_copy(eidx.at[1, wid], idx_v)

    zrow = sid * RPT
    for c in range(RPT // ZCH):
        pltpu.sync_copy(ones_v.at[pl.ds(0, ZCH)],
                        deg_sh.at[pl.ds(zrow + c * ZCH, ZCH)])
    _fill_rows(ones_v, 1.0)
    plsc.subcore_barrier()

    def add_chunk(j, _):
        pltpu.sync_copy(ones_v, deg_sh.at[idx_v.at[j]], add=True)
        return 0

    lax.fori_loop(0, NCHUNK, add_chunk, 0)
    plsc.subcore_barrier()

    pltpu.sync_copy(deg_sh.at[pl.ds(zrow, RPT)],
                    deg_out.at[cid, pl.ds(zrow, RPT)])


def _sc_edge_acc(h0, h1, eidx, acc_out, isrc_v, idst_v, rows0_v, rows1_v,
                 acc_sh, sem0, sem1):
    cid = lax.axis_index("c")
    sid = lax.axis_index("s")

    rows = (rows0_v, rows1_v)
    sems = (sem0, sem1)
    zrow = sid * RPT

    _fill_rows(rows0_v, 0.0)
    for c in range(RPT // ZCH):
        pltpu.sync_copy(rows0_v.at[pl.ds(0, ZCH)],
                        acc_sh.at[pl.ds(zrow + c * ZCH, ZCH)])
    plsc.subcore_barrier()

    for w2 in range(2):
        pltpu.sync_copy(eidx.at[0, 2 * sid + w2], isrc_v)
        pltpu.sync_copy(eidx.at[1, 2 * sid + w2], idst_v)

        for c in range(NC):
            table = h0 if c == 0 else h1

            @pl.when(cid == c)
            def _():
                pltpu.async_copy(table.at[isrc_v.at[0]], rows[0], sems[0])

                def pair(g, _):
                    for b in range(2):
                        j = 2 * g + b
                        nxt = j + 1
                        pltpu.make_async_copy(table.at[isrc_v.at[j]],
                                              rows[b], sems[b]).wait()

                        @pl.when(nxt < NCHUNK)
                        def _():
                            pltpu.async_copy(table.at[isrc_v.at[nxt]],
                                             rows[1 - b], sems[1 - b])

                        pltpu.sync_copy(rows[b],
                                        acc_sh.at[idst_v.at[j]], add=True)
                    return 0

                lax.fori_loop(0, NCHUNK // 2, pair, 0)

    plsc.subcore_barrier()
    pltpu.sync_copy(acc_sh.at[pl.ds(zrow, RPT)],
                    acc_out.at[cid, pl.ds(zrow, RPT)])


def _tc_gcn_mm_scale(x_ref, w_ref, degp_ref, h0_ref, h1_ref):
    h = lax.dot_general(
        x_ref[...], w_ref[...], (((1,), (1,)), ((), ())),
        preferred_element_type=jnp.float32)
    deg = 1.0 + degp_ref[0, :, 0] + degp_ref[1, :, 0]
    dinv = lax.rsqrt(deg)
    hs = h * dinv[:, None]
    h0_ref[...] = hs[:, :DH]
    h1_ref[...] = hs[:, DH:]


def _tc_tail(x_ref, h0_ref, h1_ref, acc_ref, degp_ref, a_ref, bias_ref,
             linw_ref, linb_ref, o_ref):
    deg = 1.0 + degp_ref[0, :, 0] + degp_ref[1, :, 0]
    dinv = lax.rsqrt(deg)
    acc = jnp.concatenate([acc_ref[0], acc_ref[1]], axis=-1)
    hs = jnp.concatenate([h0_ref[...], h1_ref[...]], axis=-1)
    agg = dinv[:, None] * (acc + hs)
    x = x_ref[...]
    pre = lax.dot_general(x, a_ref[...], (((1,), (1,)), ((), ())),
                          preferred_element_type=jnp.float32)
    pre = pre + agg + bias_ref[...]
    x2 = x + EPSILON * jnp.tanh(pre)
    x3 = jnp.where(x2 > 0, x2, jnp.exp(jnp.minimum(x2, 0.0)) - 1.0)
    o_ref[...] = lax.dot_general(x3, linw_ref[...], (((1,), (1,)), ((), ())),
                                 preferred_element_type=jnp.float32) + linb_ref[...]


def kernel(x, edge_index, W, bias, W_gcn, lin_W, lin_b):
    eidx = edge_index.astype(jnp.int32).reshape(2, NW, NCHUNK, CH)
    A = W - W.T - GAMMA * jnp.eye(D, dtype=W.dtype)
    bias2 = bias.reshape(1, D)
    linb2 = lin_b.reshape(1, D)

    mesh = plsc.VectorSubcoreMesh(core_axis_name="c", subcore_axis_name="s",
                                  num_cores=NC, num_subcores=NS)

    deg_kernel = pl.kernel(
        _sc_deg,
        out_type=jax.ShapeDtypeStruct((NC, NP, 16), jnp.float32),
        mesh=mesh,
        scratch_types=[
            pltpu.VMEM((CH, 16), jnp.float32),
            pltpu.VMEM((NCHUNK, CH), jnp.int32),
            pltpu.VMEM_SHARED((NP, 16), jnp.float32),
        ],
    )
    degp = deg_kernel(eidx)

    h0, h1 = pl.pallas_call(
        _tc_gcn_mm_scale,
        grid=(NBLK,),
        in_specs=[
            pl.BlockSpec((RB, D), lambda i: (i, 0)),
            pl.BlockSpec((D, D), lambda i: (0, 0)),
            pl.BlockSpec((NC, RB, 16), lambda i: (0, i, 0)),
        ],
        out_specs=[
            pl.BlockSpec((RB, DH), lambda i: (i, 0)),
            pl.BlockSpec((RB, DH), lambda i: (i, 0)),
        ],
        out_shape=[
            jax.ShapeDtypeStruct((N, DH), jnp.float32),
            jax.ShapeDtypeStruct((N, DH), jnp.float32),
        ],
    )(x, W_gcn, degp)

    acc_kernel = pl.kernel(
        _sc_edge_acc,
        out_type=jax.ShapeDtypeStruct((NC, NP, DH), jnp.float32),
        mesh=mesh,
        scratch_types=[
            pltpu.VMEM((NCHUNK, CH), jnp.int32),
            pltpu.VMEM((NCHUNK, CH), jnp.int32),
            pltpu.VMEM((CH, DH), jnp.float32),
            pltpu.VMEM((CH, DH), jnp.float32),
            pltpu.VMEM_SHARED((NP, DH), jnp.float32),
            pltpu.SemaphoreType.DMA,
            pltpu.SemaphoreType.DMA,
        ],
    )
    accp = acc_kernel(h0, h1, eidx)

    out = pl.pallas_call(
        _tc_tail,
        grid=(NBLK,),
        in_specs=[
            pl.BlockSpec((RB, D), lambda i: (i, 0)),
            pl.BlockSpec((RB, DH), lambda i: (i, 0)),
            pl.BlockSpec((RB, DH), lambda i: (i, 0)),
            pl.BlockSpec((NC, RB, DH), lambda i: (0, i, 0)),
            pl.BlockSpec((NC, RB, 16), lambda i: (0, i, 0)),
            pl.BlockSpec((D, D), lambda i: (0, 0)),
            pl.BlockSpec((1, D), lambda i: (0, 0)),
            pl.BlockSpec((D, D), lambda i: (0, 0)),
            pl.BlockSpec((1, D), lambda i: (0, 0)),
        ],
        out_specs=pl.BlockSpec((RB, D), lambda i: (i, 0)),
        out_shape=jax.ShapeDtypeStruct((N, D), jnp.float32),
    )(x, h0, h1, accp, degp, A, bias2, lin_W, linb2)

    return out
